# baseline (speedup 1.0000x reference)
import functools

import jax
import jax.numpy as jnp
from jax import lax
from jax.experimental import pallas as pl
from jax.experimental.pallas import tpu as pltpu
from jax.experimental.pallas import tpu_sc as plsc

N_CAT = 26
VOCAB = 1002
EMB = 250
N_NUM = 64
D_MODEL = 128
B = 1024
H = 20
N = B * H
VPAD = 1008

NC = 2
NS = 16
NW = NC * NS
ROWS_PER_W = N // NW
GS = 4
GSI = GS * N_CAT
NG = ROWS_PER_W // GS
LANES = 16


def _fold_body(t_ref, w_ref, o_ref):
    c = pl.program_id(0)
    w = w_ref[pl.ds(c * EMB, EMB), :]
    o_ref[0, :VOCAB] = jnp.dot(t_ref[0], w,
                               preferred_element_type=jnp.float32)
    o_ref[0, VOCAB:] = jnp.zeros((VPAD - VOCAB, D_MODEL), jnp.float32)


def _fold_tables(tables, w):
    return pl.pallas_call(
        _fold_body,
        grid=(N_CAT,),
        in_specs=[
            pl.BlockSpec((1, VOCAB, EMB), lambda c: (c, 0, 0)),
            pl.BlockSpec((N_CAT * EMB + N_NUM, D_MODEL), lambda c: (0, 0)),
        ],
        out_specs=pl.BlockSpec((1, VPAD, D_MODEL), lambda c: (c, 0, 0)),
        out_shape=jax.ShapeDtypeStruct((N_CAT, VPAD, D_MODEL), jnp.float32),
    )(tables, w)


NBUF = 4
OBLK = NG // NBUF
FBLK = 10
FROWS = FBLK * NBUF * GS


def _bag_body(t2_hbm, idx_hbm, out_hbm, idx_v, gbuf, obuf,
              sem0, sem1, sem2, sem3):
    wid = lax.axis_index("s") * NC + lax.axis_index("c")
    base = wid * ROWS_PER_W
    sems = (sem0, sem1, sem2, sem3)
    pltpu.sync_copy(idx_hbm.at[wid], idx_v)

    def fire(slot, g):
        pltpu.async_copy(t2_hbm.at[idx_v.at[pl.ds(g * GSI, GSI)]],
                         gbuf.at[slot], sems[slot])

    for bslot in range(NBUF):
        fire(bslot, bslot)

    def outer(blk, carry):
        fi = blk // FBLK
        frow = (blk % FBLK) * (NBUF * GS)
        for bslot in range(NBUF):
            g = blk * NBUF + bslot
            pltpu.make_async_copy(
                t2_hbm.at[idx_v.at[pl.ds(0, GSI)]], gbuf.at[bslot],
                sems[bslot]).wait()
            def tree_store(vals, row, sl):
                while len(vals) > 1:
                    vals = [vals[i] + vals[i + 1]
                            for i in range(0, len(vals) - 1, 2)] + (
                                [vals[-1]] if len(vals) % 2 else [])
                obuf[row, sl] = vals[0]

            prev = None
            for s in range(GS):
                for l in range(D_MODEL // LANES):
                    sl = pl.ds(l * LANES, LANES)
                    vals = [gbuf[bslot, s * N_CAT + c, sl]
                            for c in range(N_CAT)]
                    if prev is not None:
                        tree_store(*prev)
                    prev = (vals, frow + bslot * GS + s, sl)
            tree_store(*prev)

            @pl.when(blk < OBLK - 1)
            def _():
                fire(bslot, g + NBUF)

        @pl.when(blk % FBLK == FBLK - 1)
        def _():
            pltpu.sync_copy(obuf, out_hbm.at[pl.ds(base + fi * FROWS, FROWS)])

        return carry

    lax.fori_loop(0, OBLK, outer, 0)


def _bag_gather(t2_flat, idx):
    mesh = plsc.VectorSubcoreMesh(
        core_axis_name="c", subcore_axis_name="s", num_cores=NC, num_subcores=NS
    )
    f = pl.kernel(
        _bag_body,
        out_type=jax.ShapeDtypeStruct((N, D_MODEL), jnp.float32),
        mesh=mesh,
        compiler_params=pltpu.CompilerParams(needs_layout_passes=False),
        scratch_types=[
            pltpu.VMEM((NG * GSI,), jnp.int32),
            pltpu.VMEM((NBUF, GSI, D_MODEL), jnp.float32),
            pltpu.VMEM((FROWS, D_MODEL), jnp.float32),
            pltpu.SemaphoreType.DMA,
            pltpu.SemaphoreType.DMA,
            pltpu.SemaphoreType.DMA,
            pltpu.SemaphoreType.DMA,
        ],
    )
    return f(t2_flat, idx)


def _num_body(n_ref, wn_ref, b_ref, o_ref):
    n2 = n_ref[...].reshape(o_ref.shape[0], N_NUM)
    o_ref[...] = jnp.dot(n2, wn_ref[...],
                         preferred_element_type=jnp.float32) + b_ref[...]


def _nummat(num3, w_num, b, rbb=128):
    rb = rbb * H
    return pl.pallas_call(
        _num_body,
        grid=(B // rbb,),
        in_specs=[
            pl.BlockSpec((rbb, H, N_NUM), lambda i: (i, 0, 0)),
            pl.BlockSpec((N_NUM, D_MODEL), lambda i: (0, 0)),
            pl.BlockSpec((1, D_MODEL), lambda i: (0, 0)),
        ],
        out_specs=pl.BlockSpec((rb, D_MODEL), lambda i: (i, 0)),
        out_shape=jax.ShapeDtypeStruct((N, D_MODEL), jnp.float32),
    )(num3, w_num, b.reshape(1, D_MODEL))


def _fin_body(s_ref, p_ref, g_ref, be_ref, o_ref):
    h = s_ref[...] + p_ref[...]
    mu = jnp.mean(h, axis=1, keepdims=True)
    d = h - mu
    var = jnp.mean(d * d, axis=1, keepdims=True)
    y = d * lax.rsqrt(var + 1e-5) * g_ref[...] + be_ref[...]
    o_ref[...] = jnp.maximum(y, 0.0).reshape(o_ref.shape)


def _finalize(s, p, gamma, beta, rbb=32):
    rb = rbb * H
    return pl.pallas_call(
        _fin_body,
        grid=(N // rb,),
        in_specs=[
            pl.BlockSpec((rb, D_MODEL), lambda i: (i, 0)),
            pl.BlockSpec((rb, D_MODEL), lambda i: (i, 0)),
            pl.BlockSpec((1, D_MODEL), lambda i: (0, 0)),
            pl.BlockSpec((1, D_MODEL), lambda i: (0, 0)),
        ],
        out_specs=pl.BlockSpec((rbb, H, D_MODEL), lambda i: (i, 0, 0)),
        out_shape=jax.ShapeDtypeStruct((B, H, D_MODEL), jnp.float32),
    )(s, p, gamma.reshape(1, D_MODEL), beta.reshape(1, D_MODEL))


def kernel(cat, num, tables, W, b, gamma, beta):
    w_num = W[N_CAT * EMB:]

    t2 = _fold_tables(tables, W)
    t2_flat = t2.reshape(N_CAT * VPAD, D_MODEL)

    flat_idx = cat.reshape(N, N_CAT) + (jnp.arange(N_CAT, dtype=jnp.int32) * VPAD)
    idx = flat_idx.reshape(NW, NG * GSI)

    p = _nummat(num, w_num, b)
    s = _bag_gather(t2_flat, idx)
    return _finalize(s, p, gamma, beta)


# speedup vs baseline: 1.0275x; 1.0275x over previous
import functools

import jax
import jax.numpy as jnp
from jax import lax
from jax.experimental import pallas as pl
from jax.experimental.pallas import tpu as pltpu
from jax.experimental.pallas import tpu_sc as plsc

N_CAT = 26
VOCAB = 1002
EMB = 250
N_NUM = 64
D_MODEL = 128
B = 1024
H = 20
N = B * H
VPAD = 1008

NC = 2
NS = 16
NW = NC * NS
ROWS_PER_W = N // NW
GS = 4
GSI = GS * N_CAT
NG = ROWS_PER_W // GS
LANES = 16


def _fold_body(t_ref, w_ref, o_ref):
    o_ref[0, :VOCAB] = jnp.dot(t_ref[0], w_ref[0],
                               preferred_element_type=jnp.float32)
    o_ref[0, VOCAB:] = jnp.zeros((VPAD - VOCAB, D_MODEL), jnp.float32)


def _fold_tables(tables, w_emb):
    return pl.pallas_call(
        _fold_body,
        grid=(N_CAT,),
        in_specs=[
            pl.BlockSpec((1, VOCAB, EMB), lambda c: (c, 0, 0)),
            pl.BlockSpec((1, EMB, D_MODEL), lambda c: (c, 0, 0)),
        ],
        out_specs=pl.BlockSpec((1, VPAD, D_MODEL), lambda c: (c, 0, 0)),
        out_shape=jax.ShapeDtypeStruct((N_CAT, VPAD, D_MODEL), jnp.float32),
    )(tables, w_emb)


NBUF = 4
OBLK = NG // NBUF
FBLK = 10
FROWS = FBLK * NBUF * GS


def _bag_body(t2_hbm, idx_hbm, out_hbm, idx_v, gbuf, obuf,
              sem0, sem1, sem2, sem3):
    wid = lax.axis_index("s") * NC + lax.axis_index("c")
    base = wid * ROWS_PER_W
    sems = (sem0, sem1, sem2, sem3)
    pltpu.sync_copy(idx_hbm.at[wid], idx_v)

    def fire(slot, g):
        pltpu.async_copy(t2_hbm.at[idx_v.at[pl.ds(g * GSI, GSI)]],
                         gbuf.at[slot], sems[slot])

    for bslot in range(NBUF):
        fire(bslot, bslot)

    def outer(blk, carry):
        fi = blk // FBLK
        frow = (blk % FBLK) * (NBUF * GS)
        for bslot in range(NBUF):
            g = blk * NBUF + bslot
            pltpu.make_async_copy(
                t2_hbm.at[idx_v.at[pl.ds(0, GSI)]], gbuf.at[bslot],
                sems[bslot]).wait()
            def tree_store(vals, row, sl):
                while len(vals) > 1:
                    vals = [vals[i] + vals[i + 1]
                            for i in range(0, len(vals) - 1, 2)] + (
                                [vals[-1]] if len(vals) % 2 else [])
                obuf[row, sl] = vals[0]

            prev = None
            for s in range(GS):
                for l in range(D_MODEL // LANES):
                    sl = pl.ds(l * LANES, LANES)
                    vals = [gbuf[bslot, s * N_CAT + c, sl]
                            for c in range(N_CAT)]
                    if prev is not None:
                        tree_store(*prev)
                    prev = (vals, frow + bslot * GS + s, sl)
            tree_store(*prev)

            @pl.when(blk < OBLK - 1)
            def _():
                fire(bslot, g + NBUF)

        @pl.when(blk % FBLK == FBLK - 1)
        def _():
            pltpu.sync_copy(obuf, out_hbm.at[pl.ds(base + fi * FROWS, FROWS)])

        return carry

    lax.fori_loop(0, OBLK, outer, 0)


def _bag_gather(t2_flat, idx):
    mesh = plsc.VectorSubcoreMesh(
        core_axis_name="c", subcore_axis_name="s", num_cores=NC, num_subcores=NS
    )
    f = pl.kernel(
        _bag_body,
        out_type=jax.ShapeDtypeStruct((N, D_MODEL), jnp.float32),
        mesh=mesh,
        compiler_params=pltpu.CompilerParams(needs_layout_passes=False),
        scratch_types=[
            pltpu.VMEM((NG * GSI,), jnp.int32),
            pltpu.VMEM((NBUF, GSI, D_MODEL), jnp.float32),
            pltpu.VMEM((FROWS, D_MODEL), jnp.float32),
            pltpu.SemaphoreType.DMA,
            pltpu.SemaphoreType.DMA,
            pltpu.SemaphoreType.DMA,
            pltpu.SemaphoreType.DMA,
        ],
    )
    return f(t2_flat, idx)


def _num_body(n_ref, wn_ref, b_ref, o_ref):
    n2 = n_ref[...].reshape(o_ref.shape[0], N_NUM)
    o_ref[...] = jnp.dot(n2, wn_ref[...],
                         preferred_element_type=jnp.float32) + b_ref[...]


def _nummat(num3, w_num, b, rbb=128):
    rb = rbb * H
    return pl.pallas_call(
        _num_body,
        grid=(B // rbb,),
        in_specs=[
            pl.BlockSpec((rbb, H, N_NUM), lambda i: (i, 0, 0)),
            pl.BlockSpec((N_NUM, D_MODEL), lambda i: (0, 0)),
            pl.BlockSpec((1, D_MODEL), lambda i: (0, 0)),
        ],
        out_specs=pl.BlockSpec((rb, D_MODEL), lambda i: (i, 0)),
        out_shape=jax.ShapeDtypeStruct((N, D_MODEL), jnp.float32),
    )(num3, w_num, b.reshape(1, D_MODEL))


def _fin_body(s_ref, p_ref, g_ref, be_ref, o_ref):
    h = s_ref[...] + p_ref[...]
    mu = jnp.mean(h, axis=1, keepdims=True)
    d = h - mu
    var = jnp.mean(d * d, axis=1, keepdims=True)
    y = d * lax.rsqrt(var + 1e-5) * g_ref[...] + be_ref[...]
    o_ref[...] = jnp.maximum(y, 0.0).reshape(o_ref.shape)


def _finalize(s, p, gamma, beta, rbb=64):
    rb = rbb * H
    return pl.pallas_call(
        _fin_body,
        grid=(N // rb,),
        in_specs=[
            pl.BlockSpec((rb, D_MODEL), lambda i: (i, 0)),
            pl.BlockSpec((rb, D_MODEL), lambda i: (i, 0)),
            pl.BlockSpec((1, D_MODEL), lambda i: (0, 0)),
            pl.BlockSpec((1, D_MODEL), lambda i: (0, 0)),
        ],
        out_specs=pl.BlockSpec((rbb, H, D_MODEL), lambda i: (i, 0, 0)),
        out_shape=jax.ShapeDtypeStruct((B, H, D_MODEL), jnp.float32),
    )(s, p, gamma.reshape(1, D_MODEL), beta.reshape(1, D_MODEL))


def kernel(cat, num, tables, W, b, gamma, beta):
    w_emb = W[: N_CAT * EMB].reshape(N_CAT, EMB, D_MODEL)
    w_num = W[N_CAT * EMB:]

    t2 = _fold_tables(tables, w_emb)
    t2_flat = t2.reshape(N_CAT * VPAD, D_MODEL)

    flat_idx = cat.reshape(N, N_CAT) + (jnp.arange(N_CAT, dtype=jnp.int32) * VPAD)
    idx = flat_idx.reshape(NW, NG * GSI)

    p = _nummat(num, w_num, b)
    s = _bag_gather(t2_flat, idx)
    return _finalize(s, p, gamma, beta)


# speedup vs baseline: 1.0532x; 1.0249x over previous
import functools

import jax
import jax.numpy as jnp
from jax import lax
from jax.experimental import pallas as pl
from jax.experimental.pallas import tpu as pltpu
from jax.experimental.pallas import tpu_sc as plsc

N_CAT = 26
VOCAB = 1002
EMB = 250
N_NUM = 64
D_MODEL = 128
B = 1024
H = 20
N = B * H
VPAD = 1008
HP = 24

NC = 2
NS = 16
NW = NC * NS
ROWS_PER_W = N // NW
GS = 4
GSI = GS * N_CAT
NG = ROWS_PER_W // GS
LANES = 16


def _fold_body(t_ref, w_ref, o_ref):
    o_ref[0, :VOCAB] = jnp.dot(t_ref[0], w_ref[0],
                               preferred_element_type=jnp.float32)
    o_ref[0, VOCAB:] = jnp.zeros((VPAD - VOCAB, D_MODEL), jnp.float32)


def _fold_tables(tables, w_emb):
    return pl.pallas_call(
        _fold_body,
        grid=(N_CAT,),
        in_specs=[
            pl.BlockSpec((1, VOCAB, EMB), lambda c: (c, 0, 0)),
            pl.BlockSpec((1, EMB, D_MODEL), lambda c: (c, 0, 0)),
        ],
        out_specs=pl.BlockSpec((1, VPAD, D_MODEL), lambda c: (c, 0, 0)),
        out_shape=jax.ShapeDtypeStruct((N_CAT, VPAD, D_MODEL), jnp.float32),
    )(tables, w_emb)


NBUF = 4
OBLK = NG // NBUF
FBLK = 10
FROWS = FBLK * NBUF * GS


def _bag_body(t2_hbm, idx_hbm, out_hbm, idx_v, gbuf, obuf,
              sem0, sem1, sem2, sem3):
    wid = lax.axis_index("s") * NC + lax.axis_index("c")
    base = wid * ROWS_PER_W
    sems = (sem0, sem1, sem2, sem3)
    pltpu.sync_copy(idx_hbm.at[wid], idx_v)

    def fire(slot, g):
        pltpu.async_copy(t2_hbm.at[idx_v.at[pl.ds(g * GSI, GSI)]],
                         gbuf.at[slot], sems[slot])

    for bslot in range(NBUF):
        fire(bslot, bslot)

    def outer(blk, carry):
        fi = blk // FBLK
        frow = (blk % FBLK) * (NBUF * GS)
        for bslot in range(NBUF):
            g = blk * NBUF + bslot
            pltpu.make_async_copy(
                t2_hbm.at[idx_v.at[pl.ds(0, GSI)]], gbuf.at[bslot],
                sems[bslot]).wait()
            def tree_store(vals, row, sl):
                while len(vals) > 1:
                    vals = [vals[i] + vals[i + 1]
                            for i in range(0, len(vals) - 1, 2)] + (
                                [vals[-1]] if len(vals) % 2 else [])
                obuf[row, sl] = vals[0]

            prev = None
            for s in range(GS):
                for l in range(D_MODEL // LANES):
                    sl = pl.ds(l * LANES, LANES)
                    vals = [gbuf[bslot, s * N_CAT + c, sl]
                            for c in range(N_CAT)]
                    if prev is not None:
                        tree_store(*prev)
                    n = frow + bslot * GS + s
                    prev = (vals, (n // H) * HP + n % H, sl)
            tree_store(*prev)

            @pl.when(blk < OBLK - 1)
            def _():
                fire(bslot, g + NBUF)

        @pl.when(blk % FBLK == FBLK - 1)
        def _():
            pltpu.sync_copy(
                obuf,
                out_hbm.at[pl.ds(
                    (wid * (ROWS_PER_W // H) + fi * (FROWS // H)) * HP,
                    (FROWS // H) * HP)])

        return carry

    lax.fori_loop(0, OBLK, outer, 0)


def _bag_gather(t2_flat, idx):
    mesh = plsc.VectorSubcoreMesh(
        core_axis_name="c", subcore_axis_name="s", num_cores=NC, num_subcores=NS
    )
    f = pl.kernel(
        _bag_body,
        out_type=jax.ShapeDtypeStruct((B * HP, D_MODEL), jnp.float32),
        mesh=mesh,
        compiler_params=pltpu.CompilerParams(needs_layout_passes=False),
        scratch_types=[
            pltpu.VMEM((NG * GSI,), jnp.int32),
            pltpu.VMEM((NBUF, GSI, D_MODEL), jnp.float32),
            pltpu.VMEM(((FROWS // H) * HP, D_MODEL), jnp.float32),
            pltpu.SemaphoreType.DMA,
            pltpu.SemaphoreType.DMA,
            pltpu.SemaphoreType.DMA,
            pltpu.SemaphoreType.DMA,
        ],
    )
    return f(t2_flat, idx)


def _num_body(n_ref, wn_ref, b_ref, o_ref):
    rbb = n_ref.shape[0]
    n2 = n_ref[...].reshape(rbb * H, N_NUM)
    p = jnp.dot(n2, wn_ref[...],
                preferred_element_type=jnp.float32) + b_ref[...]
    o_ref[:, :H] = p.reshape(rbb, H, D_MODEL)
    o_ref[:, H:] = jnp.zeros((rbb, HP - H, D_MODEL), jnp.float32)


def _nummat(num3, w_num, b, rbb=128):
    return pl.pallas_call(
        _num_body,
        grid=(B // rbb,),
        in_specs=[
            pl.BlockSpec((rbb, H, N_NUM), lambda i: (i, 0, 0)),
            pl.BlockSpec((N_NUM, D_MODEL), lambda i: (0, 0)),
            pl.BlockSpec((1, D_MODEL), lambda i: (0, 0)),
        ],
        out_specs=pl.BlockSpec((rbb, HP, D_MODEL), lambda i: (i, 0, 0)),
        out_shape=jax.ShapeDtypeStruct((B, HP, D_MODEL), jnp.float32),
    )(num3, w_num, b.reshape(1, D_MODEL))


def _fin_body(s_ref, p_ref, g_ref, be_ref, o_ref):
    h = s_ref[:, :H] + p_ref[:, :H]
    mu = jnp.mean(h, axis=2, keepdims=True)
    d = h - mu
    var = jnp.mean(d * d, axis=2, keepdims=True)
    y = d * lax.rsqrt(var + 1e-5) * g_ref[...] + be_ref[...]
    o_ref[...] = jnp.maximum(y, 0.0).transpose(1, 0, 2)


def _finalize(s3, p3, gamma, beta, rbb=64):
    return pl.pallas_call(
        _fin_body,
        grid=(B // rbb,),
        in_specs=[
            pl.BlockSpec((rbb, HP, D_MODEL), lambda i: (i, 0, 0)),
            pl.BlockSpec((rbb, HP, D_MODEL), lambda i: (i, 0, 0)),
            pl.BlockSpec((1, 1, D_MODEL), lambda i: (0, 0, 0)),
            pl.BlockSpec((1, 1, D_MODEL), lambda i: (0, 0, 0)),
        ],
        out_specs=pl.BlockSpec((H, rbb, D_MODEL), lambda i: (0, i, 0)),
        out_shape=jax.ShapeDtypeStruct((H, B, D_MODEL), jnp.float32),
    )(s3, p3, gamma.reshape(1, 1, D_MODEL), beta.reshape(1, 1, D_MODEL))


def kernel(cat, num, tables, W, b, gamma, beta):
    w_emb = W[: N_CAT * EMB].reshape(N_CAT, EMB, D_MODEL)
    w_num = W[N_CAT * EMB:]

    t2 = _fold_tables(tables, w_emb)
    t2_flat = t2.reshape(N_CAT * VPAD, D_MODEL)

    flat_idx = cat.reshape(N, N_CAT) + (jnp.arange(N_CAT, dtype=jnp.int32) * VPAD)
    idx = flat_idx.reshape(NW, NG * GSI)

    p3 = _nummat(num, w_num, b)
    s3 = _bag_gather(t2_flat, idx).reshape(B, HP, D_MODEL)
    out_t = _finalize(s3, p3, gamma, beta)
    return jnp.transpose(out_t, (1, 0, 2))


# speedup vs baseline: 1.0588x; 1.0054x over previous
import functools

import jax
import jax.numpy as jnp
from jax import lax
from jax.experimental import pallas as pl
from jax.experimental.pallas import tpu as pltpu
from jax.experimental.pallas import tpu_sc as plsc

N_CAT = 26
VOCAB = 1002
EMB = 250
N_NUM = 64
D_MODEL = 128
B = 1024
H = 20
N = B * H
VPAD = 1008
HP = 24

NC = 2
NS = 16
NW = NC * NS
ROWS_PER_W = N // NW
GS = 4
GSI = GS * N_CAT
NG = ROWS_PER_W // GS
LANES = 16


def _fold_body(t_ref, w_ref, o_ref):
    o_ref[0, :VOCAB] = jnp.dot(t_ref[0], w_ref[0],
                               preferred_element_type=jnp.float32)
    o_ref[0, VOCAB:] = jnp.zeros((VPAD - VOCAB, D_MODEL), jnp.float32)


def _fold_tables(tables, w_emb):
    return pl.pallas_call(
        _fold_body,
        grid=(N_CAT,),
        in_specs=[
            pl.BlockSpec((1, VOCAB, EMB), lambda c: (c, 0, 0)),
            pl.BlockSpec((1, EMB, D_MODEL), lambda c: (c, 0, 0)),
        ],
        out_specs=pl.BlockSpec((1, VPAD, D_MODEL), lambda c: (c, 0, 0)),
        out_shape=jax.ShapeDtypeStruct((N_CAT, VPAD, D_MODEL), jnp.float32),
    )(tables, w_emb)


NBUF = 4
OBLK = NG // NBUF
FBLK = 10
FROWS = FBLK * NBUF * GS


def _bag_body(t2_hbm, idx_hbm, out_hbm, idx_v, gbuf, obuf,
              sem0, sem1, sem2, sem3):
    wid = lax.axis_index("s") * NC + lax.axis_index("c")
    base = wid * ROWS_PER_W
    sems = (sem0, sem1, sem2, sem3)
    pltpu.sync_copy(idx_hbm.at[wid], idx_v)

    def fire(slot, g):
        pltpu.async_copy(t2_hbm.at[idx_v.at[pl.ds(g * GSI, GSI)]],
                         gbuf.at[slot], sems[slot])

    for bslot in range(NBUF):
        fire(bslot, bslot)

    def outer(blk, carry):
        fi = blk // FBLK
        frow = (blk % FBLK) * (NBUF * GS)
        for bslot in range(NBUF):
            g = blk * NBUF + bslot
            pltpu.make_async_copy(
                t2_hbm.at[idx_v.at[pl.ds(0, GSI)]], gbuf.at[bslot],
                sems[bslot]).wait()
            def tree_store(vals, row, sl):
                while len(vals) > 1:
                    vals = [vals[i] + vals[i + 1]
                            for i in range(0, len(vals) - 1, 2)] + (
                                [vals[-1]] if len(vals) % 2 else [])
                obuf[row, sl] = vals[0]

            prev = None
            for s in range(GS):
                for l in range(D_MODEL // LANES):
                    sl = pl.ds(l * LANES, LANES)
                    vals = [gbuf[bslot, s * N_CAT + c, sl]
                            for c in range(N_CAT)]
                    if prev is not None:
                        tree_store(*prev)
                    n = frow + bslot * GS + s
                    prev = (vals, (n // H) * HP + n % H, sl)
            tree_store(*prev)

            @pl.when(blk < OBLK - 1)
            def _():
                fire(bslot, g + NBUF)

        @pl.when(blk % FBLK == FBLK - 1)
        def _():
            pltpu.sync_copy(
                obuf,
                out_hbm.at[pl.ds(
                    (wid * (ROWS_PER_W // H) + fi * (FROWS // H)) * HP,
                    (FROWS // H) * HP)])

        return carry

    lax.fori_loop(0, OBLK, outer, 0)


def _bag_gather(t2_flat, idx):
    mesh = plsc.VectorSubcoreMesh(
        core_axis_name="c", subcore_axis_name="s", num_cores=NC, num_subcores=NS
    )
    f = pl.kernel(
        _bag_body,
        out_type=jax.ShapeDtypeStruct((B * HP, D_MODEL), jnp.float32),
        mesh=mesh,
        compiler_params=pltpu.CompilerParams(needs_layout_passes=False),
        scratch_types=[
            pltpu.VMEM((NG * GSI,), jnp.int32),
            pltpu.VMEM((NBUF, GSI, D_MODEL), jnp.float32),
            pltpu.VMEM(((FROWS // H) * HP, D_MODEL), jnp.float32),
            pltpu.SemaphoreType.DMA,
            pltpu.SemaphoreType.DMA,
            pltpu.SemaphoreType.DMA,
            pltpu.SemaphoreType.DMA,
        ],
    )
    return f(t2_flat, idx)


def _num_body(n_ref, wn_ref, b_ref, o_ref):
    rbb = n_ref.shape[0]
    n2 = n_ref[...].reshape(rbb * H, N_NUM)
    p = jnp.dot(n2, wn_ref[...],
                preferred_element_type=jnp.float32) + b_ref[...]
    o_ref[:, :H] = p.reshape(rbb, H, D_MODEL)
    o_ref[:, H:] = jnp.zeros((rbb, HP - H, D_MODEL), jnp.float32)


def _nummat(num3, w_num, b, rbb=128):
    return pl.pallas_call(
        _num_body,
        grid=(B // rbb,),
        in_specs=[
            pl.BlockSpec((rbb, H, N_NUM), lambda i: (i, 0, 0)),
            pl.BlockSpec((N_NUM, D_MODEL), lambda i: (0, 0)),
            pl.BlockSpec((1, D_MODEL), lambda i: (0, 0)),
        ],
        out_specs=pl.BlockSpec((rbb, HP, D_MODEL), lambda i: (i, 0, 0)),
        out_shape=jax.ShapeDtypeStruct((B, HP, D_MODEL), jnp.float32),
    )(num3, w_num, b.reshape(1, D_MODEL))


def _fin_body(s_ref, p_ref, g_ref, be_ref, o_ref):
    h = s_ref[:, :H] + p_ref[:, :H]
    mu = jnp.mean(h, axis=2, keepdims=True)
    d = h - mu
    var = jnp.mean(d * d, axis=2, keepdims=True)
    y = d * lax.rsqrt(var + 1e-5) * g_ref[...] + be_ref[...]
    o_ref[...] = jnp.maximum(y, 0.0).transpose(1, 0, 2)


def _finalize(s3, p3, gamma, beta, rbb=128):
    return pl.pallas_call(
        _fin_body,
        grid=(B // rbb,),
        in_specs=[
            pl.BlockSpec((rbb, HP, D_MODEL), lambda i: (i, 0, 0)),
            pl.BlockSpec((rbb, HP, D_MODEL), lambda i: (i, 0, 0)),
            pl.BlockSpec((1, 1, D_MODEL), lambda i: (0, 0, 0)),
            pl.BlockSpec((1, 1, D_MODEL), lambda i: (0, 0, 0)),
        ],
        out_specs=pl.BlockSpec((H, rbb, D_MODEL), lambda i: (0, i, 0)),
        out_shape=jax.ShapeDtypeStruct((H, B, D_MODEL), jnp.float32),
    )(s3, p3, gamma.reshape(1, 1, D_MODEL), beta.reshape(1, 1, D_MODEL))


def kernel(cat, num, tables, W, b, gamma, beta):
    w_emb = W[: N_CAT * EMB].reshape(N_CAT, EMB, D_MODEL)
    w_num = W[N_CAT * EMB:]

    t2 = _fold_tables(tables, w_emb)
    t2_flat = t2.reshape(N_CAT * VPAD, D_MODEL)

    flat_idx = cat.reshape(N, N_CAT) + (jnp.arange(N_CAT, dtype=jnp.int32) * VPAD)
    idx = flat_idx.reshape(NW, NG * GSI)

    p3 = _nummat(num, w_num, b)
    s3 = _bag_gather(t2_flat, idx).reshape(B, HP, D_MODEL)
    out_t = _finalize(s3, p3, gamma, beta)
    return jnp.transpose(out_t, (1, 0, 2))
